# baseline (speedup 1.0000x reference)
import functools

import jax
import jax.numpy as jnp
from jax import lax
from jax.experimental import pallas as pl
from jax.experimental.pallas import tpu as pltpu
from jax.experimental.pallas import tpu_sc as plsc

TOPK = 10
BETA = 2.0
NEG = -1.7e38
BIGI = 2**30
BIGF = 3.0e7


def _topk_body(nb, n_real, bn, q_ref, db_ref, outv_ref, outi_ref, runv, runi):
    j = pl.program_id(1)

    @pl.when(j == 0)
    def _init():
        runv[...] = jnp.full(runv.shape, NEG, jnp.float32)
        runi[...] = jnp.full(runi.shape, -1, jnp.int32)

    bq = q_ref.shape[0]
    tg = bn // 128
    s = lax.dot_general(q_ref[...], db_ref[...],
                        (((1,), (1,)), ((), ())),
                        preferred_element_type=jnp.float32)
    row = lax.broadcasted_iota(jnp.int32, (1, bn), 1)
    s = jnp.where(j * bn + row < n_real, s, NEG)

    lane = lax.broadcasted_iota(jnp.int32, (bq, 128), 1)
    base = j * bn + lane

    zi = jnp.zeros((bq, 128), jnp.int32)
    negf = jnp.full((bq, 128), NEG, jnp.float32)
    p1, p2, p3, p4 = negf, negf, negf, negf
    t1, t2, t3 = zi, zi, zi
    for k in range(tg):
        v = s[:, k * 128:(k + 1) * 128]
        g1 = v > p1
        g2 = v > p2
        g3 = v > p3
        g4 = v > p4
        p4 = jnp.where(g3, p3, jnp.where(g4, v, p4))
        p3 = jnp.where(g2, p2, jnp.where(g3, v, p3))
        t3 = jnp.where(g2, t2, jnp.where(g3, k, t3))
        p2 = jnp.where(g1, p1, jnp.where(g2, v, p2))
        t2 = jnp.where(g1, t1, jnp.where(g2, k, t2))
        p1 = jnp.where(g1, v, p1)
        t1 = jnp.where(g1, k, t1)
    j1 = base + t1 * 128
    j2 = base + t2 * 128
    j3 = base + t3 * 128

    x = jnp.concatenate([p1, p2, p3, runv[...]], axis=1)
    c = jnp.concatenate([j1, j2, j3, runi[...]], axis=1).astype(jnp.float32)
    newv, newi = [], []
    for _ in range(TOPK):
        m = jnp.max(x, axis=1, keepdims=True)
        ji = jnp.where(x == m, c, BIGF)
        am = jnp.min(ji, axis=1, keepdims=True)
        newv.append(m)
        newi.append(am)
        x = jnp.where(c == am, NEG, x)

    drained = ((x[:, 0:128] == NEG) & (p1 != NEG) &
               (x[:, 128:256] == NEG) & (p2 != NEG) &
               (x[:, 256:384] == NEG) & (p3 != NEG))
    flag = jnp.max(jnp.where(drained & (p4 >= newv[-1]), 1, 0))

    pad = runv.shape[1] - TOPK
    padv = [jnp.full((bq, pad), NEG, jnp.float32)]
    padi = [jnp.full((bq, pad), -1, jnp.int32)]

    @pl.when(flag == 0)
    def _fast():
        runv[...] = jnp.concatenate(newv + padv, axis=1)
        runi[...] = jnp.concatenate(
            [jnp.concatenate(newi, axis=1).astype(jnp.int32)] + padi, axis=1)

    @pl.when(flag != 0)
    def _exact():
        cols = j * bn + lax.broadcasted_iota(jnp.int32, (bq, bn), 1)
        sa = jnp.concatenate([s, runv[...]], axis=1)
        ca = jnp.concatenate(
            [cols, runi[...]], axis=1).astype(jnp.float32)
        nv, ni = [], []
        for _ in range(TOPK):
            m = jnp.max(sa, axis=1, keepdims=True)
            ji = jnp.where(sa == m, ca, BIGF)
            am = jnp.min(ji, axis=1, keepdims=True)
            nv.append(m)
            ni.append(am)
            sa = jnp.where(ca == am, NEG, sa)
        runv[...] = jnp.concatenate(nv + padv, axis=1)
        runi[...] = jnp.concatenate(
            [jnp.concatenate(ni, axis=1).astype(jnp.int32)] + padi, axis=1)

    @pl.when(j == nb - 1)
    def _flush():
        outv_ref[...] = runv[:, :TOPK]
        outi_ref[...] = runi[:, :TOPK]


def _topk(q, db_p, n_real, bq=256, bn=8192, interpret=False):
    qn, d = q.shape
    npad = db_p.shape[0]
    assert qn % bq == 0 and npad % bn == 0
    grid = (qn // bq, npad // bn)
    body = functools.partial(_topk_body, grid[1], n_real, bn)
    return pl.pallas_call(
        body,
        grid=grid,
        in_specs=[
            pl.BlockSpec((bq, d), lambda i, j: (i, 0)),
            pl.BlockSpec((bn, d), lambda i, j: (j, 0)),
        ],
        out_specs=[
            pl.BlockSpec((bq, TOPK), lambda i, j: (i, 0)),
            pl.BlockSpec((bq, TOPK), lambda i, j: (i, 0)),
        ],
        out_shape=[
            jax.ShapeDtypeStruct((qn, TOPK), jnp.float32),
            jax.ShapeDtypeStruct((qn, TOPK), jnp.int32),
        ],
        scratch_shapes=[
            pltpu.VMEM((bq, 16), jnp.float32),
            pltpu.VMEM((bq, 16), jnp.int32),
        ],
        compiler_params=pltpu.CompilerParams(
            dimension_semantics=("parallel", "arbitrary")),
        interpret=interpret,
    )(q, db_p)


def _sc_gather(table, idx):
    b, d = idx.shape[0], table.shape[1]
    info = plsc.get_sparse_core_info()
    nc, ns = info.num_cores, info.num_subcores
    nw = nc * ns
    assert b % (8 * nw) == 0
    b_per_w = b // nw
    chunk = b_per_w
    while chunk * d * 4 > 400 * 1024:
        chunk //= 2
    assert b_per_w % chunk == 0 and chunk % 8 == 0
    nchunks = b_per_w // chunk
    mesh = plsc.VectorSubcoreMesh(core_axis_name="c", subcore_axis_name="s")

    @functools.partial(
        pl.kernel, mesh=mesh,
        out_type=jax.ShapeDtypeStruct((b, d), jnp.float32),
        scratch_types=[
            pltpu.VMEM((chunk,), jnp.int32),
            pltpu.VMEM((chunk, d), jnp.float32),
            pltpu.SemaphoreType.DMA,
        ],
    )
    def k(table_hbm, idx_hbm, out_hbm, idx_v, rows_v, sem):
        wid = lax.axis_index("s") * nc + lax.axis_index("c")
        base = wid * b_per_w
        for c in range(nchunks):
            off = base + c * chunk
            pltpu.sync_copy(idx_hbm.at[pl.ds(off, chunk)], idx_v)
            pltpu.async_copy(table_hbm.at[idx_v], rows_v, sem).wait()
            pltpu.sync_copy(rows_v, out_hbm.at[pl.ds(off, chunk)])

    return k(table, idx)


def _rsqrt_nr(x):
    r = lax.rsqrt(x)
    r = r * (1.5 - 0.5 * x * r * r)
    r = r * (1.5 - 0.5 * x * r * r)
    return r


def _epilogue_body(q_ref, v1_ref, ids1_ref, s2_ref, v2_ref, rid_ref, rs_ref):
    bq = q_ref.shape[0]
    d = q_ref.shape[1]
    q = q_ref[...]
    v1 = v1_ref[...].reshape(bq, TOPK, d)
    s2 = s2_ref[...]
    v2 = v2_ref[...].reshape(bq * TOPK, TOPK, d)

    top9 = jnp.max(v1[:, : TOPK - 1, :], axis=1)
    qtk = jnp.maximum(q, top9)
    qn = qtk * _rsqrt_nr(
        jnp.maximum(jnp.sum(qtk * qtk, axis=1, keepdims=True), 1e-24))

    w = s2 * BETA
    qrep = jnp.broadcast_to(q[:, None, :], (bq, TOPK, d)).reshape(bq * TOPK, d)
    ws = jnp.sum(v2 * w[:, :, None], axis=1) + BETA * qrep
    nf = 1.0 + BETA + jnp.sum(w, axis=1, keepdims=True)
    sgn = jnp.where(nf >= 0.0, 1.0, -1.0)
    inv = _rsqrt_nr(
        jnp.maximum(jnp.sum(ws * ws, axis=1, keepdims=True), 1e-24))
    refined = ws * (sgn * inv)

    def _bf(x):
        return x.astype(jnp.bfloat16).astype(jnp.float32)

    refb = _bf(refined)
    sc = 0.5 * (jnp.sum(refb * _bf(qrep), axis=1) +
                jnp.sum(refb.reshape(bq, TOPK, d) * _bf(qn)[:, None, :],
                        axis=2).reshape(bq * TOPK))
    sc = sc.reshape(bq, TOPK)

    ids1 = ids1_ref[...]
    mio = lax.broadcasted_iota(jnp.int32, (bq, TOPK), 1)
    vals, rids = [], []
    for _ in range(3):
        m = jnp.max(sc, axis=1, keepdims=True)
        pos = jnp.min(jnp.where(sc == m, mio, BIGI), axis=1, keepdims=True)
        vals.append(m)
        rids.append(jnp.sum(jnp.where(mio == pos, ids1, 0), axis=1,
                            keepdims=True))
        sc = jnp.where(mio == pos, NEG, sc)
    rs_ref[...] = jnp.concatenate(vals, axis=1)
    rid_ref[...] = jnp.concatenate(rids, axis=1)


def _epilogue(q, v1, ids1, s2, v2, bq=128, interpret=False):
    qn, d = q.shape
    grid = (qn // bq,)
    return pl.pallas_call(
        _epilogue_body,
        grid=grid,
        in_specs=[
            pl.BlockSpec((bq, d), lambda i: (i, 0)),
            pl.BlockSpec((bq * TOPK, d), lambda i: (i, 0)),
            pl.BlockSpec((bq, TOPK), lambda i: (i, 0)),
            pl.BlockSpec((bq * TOPK, TOPK), lambda i: (i, 0)),
            pl.BlockSpec((bq * TOPK * TOPK, d), lambda i: (i, 0)),
        ],
        out_specs=[
            pl.BlockSpec((bq, 3), lambda i: (i, 0)),
            pl.BlockSpec((bq, 3), lambda i: (i, 0)),
        ],
        out_shape=[
            jax.ShapeDtypeStruct((qn, 3), jnp.int32),
            jax.ShapeDtypeStruct((qn, 3), jnp.float32),
        ],
        interpret=interpret,
    )(q, v1, ids1, s2, v2)


def kernel(query_features, db):
    n, d = db.shape
    bn = 8192
    npad = ((n + bn - 1) // bn) * bn
    db_p = jnp.concatenate(
        [db, jnp.zeros((npad - n, d), db.dtype)], axis=0)

    _, ids1 = _topk(query_features, db_p, n)
    ids1_flat = ids1.reshape(-1)
    top_m = _sc_gather(db_p, ids1_flat)

    s2, ids2 = _topk(top_m, db_p, n)
    v2 = _sc_gather(db_p, ids2.reshape(-1))

    rid, rs = _epilogue(query_features, top_m, ids1, s2, v2)
    return (rid, rs)


# speedup vs baseline: 1.1771x; 1.1771x over previous
import functools

import jax
import jax.numpy as jnp
from jax import lax
from jax.experimental import pallas as pl
from jax.experimental.pallas import tpu as pltpu
from jax.experimental.pallas import tpu_sc as plsc

TOPK = 10
BETA = 2.0
NEG = -1.7e38
BIGI = 2**30
BIGF = 3.0e7


def _topk_body(nb, n_real, bn, q_ref, db_ref, outv_ref, outi_ref, runv, runi):
    j = pl.program_id(1)

    @pl.when(j == 0)
    def _init():
        runv[...] = jnp.full(runv.shape, NEG, jnp.float32)
        runi[...] = jnp.full(runi.shape, -1, jnp.int32)

    bq = q_ref.shape[0]
    tg = bn // 128
    s = lax.dot_general(q_ref[...], db_ref[...],
                        (((1,), (1,)), ((), ())),
                        preferred_element_type=jnp.float32)
    row = lax.broadcasted_iota(jnp.int32, (1, bn), 1)
    s = jnp.where(j * bn + row < n_real, s, NEG)

    lane = lax.broadcasted_iota(jnp.int32, (bq, 128), 1)
    base = j * bn + lane

    def _level(slices):
        p = slices[0]
        for sk in slices[1:]:
            p = jnp.maximum(p, sk)
        t = jnp.full((bq, 128), tg - 1, jnp.int32)
        for k in range(tg - 1, -1, -1):
            t = jnp.where(slices[k] == p, k, t)
        nxt = [jnp.where(t == k, NEG, slices[k]) for k in range(tg)]
        return p, t, nxt

    sl = [s[:, k * 128:(k + 1) * 128] for k in range(tg)]
    p1, t1, sl2 = _level(sl)
    p2, t2, sl3 = _level(sl2)
    p3, t3, sl4 = _level(sl3)
    p4 = sl4[0]
    for sk in sl4[1:]:
        p4 = jnp.maximum(p4, sk)
    j1 = base + t1 * 128
    j2 = base + t2 * 128
    j3 = base + t3 * 128

    x = jnp.concatenate([p1, p2, p3, runv[...]], axis=1)
    c = jnp.concatenate([j1, j2, j3, runi[...]], axis=1).astype(jnp.float32)
    newv, newi = [], []
    for _ in range(TOPK):
        m = jnp.max(x, axis=1, keepdims=True)
        ji = jnp.where(x == m, c, BIGF)
        am = jnp.min(ji, axis=1, keepdims=True)
        newv.append(m)
        newi.append(am)
        x = jnp.where(c == am, NEG, x)

    drained = ((x[:, 0:128] == NEG) & (p1 != NEG) &
               (x[:, 128:256] == NEG) & (p2 != NEG) &
               (x[:, 256:384] == NEG) & (p3 != NEG))
    flag = jnp.max(jnp.where(drained & (p4 >= newv[-1]), 1, 0))

    pad = runv.shape[1] - TOPK
    padv = [jnp.full((bq, pad), NEG, jnp.float32)]
    padi = [jnp.full((bq, pad), -1, jnp.int32)]

    @pl.when(flag == 0)
    def _fast():
        runv[...] = jnp.concatenate(newv + padv, axis=1)
        runi[...] = jnp.concatenate(
            [jnp.concatenate(newi, axis=1).astype(jnp.int32)] + padi, axis=1)

    @pl.when(flag != 0)
    def _exact():
        cols = j * bn + lax.broadcasted_iota(jnp.int32, (bq, bn), 1)
        sa = jnp.concatenate([s, runv[...]], axis=1)
        ca = jnp.concatenate(
            [cols, runi[...]], axis=1).astype(jnp.float32)
        nv, ni = [], []
        for _ in range(TOPK):
            m = jnp.max(sa, axis=1, keepdims=True)
            ji = jnp.where(sa == m, ca, BIGF)
            am = jnp.min(ji, axis=1, keepdims=True)
            nv.append(m)
            ni.append(am)
            sa = jnp.where(ca == am, NEG, sa)
        runv[...] = jnp.concatenate(nv + padv, axis=1)
        runi[...] = jnp.concatenate(
            [jnp.concatenate(ni, axis=1).astype(jnp.int32)] + padi, axis=1)

    @pl.when(j == nb - 1)
    def _flush():
        outv_ref[...] = runv[:, :TOPK]
        outi_ref[...] = runi[:, :TOPK]


def _topk(q, db_p, n_real, bq=256, bn=10240, interpret=False):
    qn, d = q.shape
    npad = db_p.shape[0]
    assert qn % bq == 0 and npad % bn == 0
    grid = (qn // bq, npad // bn)
    body = functools.partial(_topk_body, grid[1], n_real, bn)
    return pl.pallas_call(
        body,
        grid=grid,
        in_specs=[
            pl.BlockSpec((bq, d), lambda i, j: (i, 0)),
            pl.BlockSpec((bn, d), lambda i, j: (j, 0)),
        ],
        out_specs=[
            pl.BlockSpec((bq, TOPK), lambda i, j: (i, 0)),
            pl.BlockSpec((bq, TOPK), lambda i, j: (i, 0)),
        ],
        out_shape=[
            jax.ShapeDtypeStruct((qn, TOPK), jnp.float32),
            jax.ShapeDtypeStruct((qn, TOPK), jnp.int32),
        ],
        scratch_shapes=[
            pltpu.VMEM((bq, 16), jnp.float32),
            pltpu.VMEM((bq, 16), jnp.int32),
        ],
        compiler_params=pltpu.CompilerParams(
            dimension_semantics=("parallel", "arbitrary")),
        interpret=interpret,
    )(q, db_p)


def _sc_gather(table, idx):
    b, d = idx.shape[0], table.shape[1]
    info = plsc.get_sparse_core_info()
    nc, ns = info.num_cores, info.num_subcores
    nw = nc * ns
    assert b % (8 * nw) == 0
    b_per_w = b // nw
    chunk = b_per_w
    while chunk * d * 4 > 400 * 1024:
        chunk //= 2
    assert b_per_w % chunk == 0 and chunk % 8 == 0
    nchunks = b_per_w // chunk
    mesh = plsc.VectorSubcoreMesh(core_axis_name="c", subcore_axis_name="s")

    @functools.partial(
        pl.kernel, mesh=mesh,
        out_type=jax.ShapeDtypeStruct((b, d), jnp.float32),
        scratch_types=[
            pltpu.VMEM((chunk,), jnp.int32),
            pltpu.VMEM((chunk, d), jnp.float32),
            pltpu.SemaphoreType.DMA,
        ],
    )
    def k(table_hbm, idx_hbm, out_hbm, idx_v, rows_v, sem):
        wid = lax.axis_index("s") * nc + lax.axis_index("c")
        base = wid * b_per_w
        for c in range(nchunks):
            off = base + c * chunk
            pltpu.sync_copy(idx_hbm.at[pl.ds(off, chunk)], idx_v)
            pltpu.async_copy(table_hbm.at[idx_v], rows_v, sem).wait()
            pltpu.sync_copy(rows_v, out_hbm.at[pl.ds(off, chunk)])

    return k(table, idx)


def _rsqrt_nr(x):
    r = lax.rsqrt(x)
    r = r * (1.5 - 0.5 * x * r * r)
    r = r * (1.5 - 0.5 * x * r * r)
    return r


def _epilogue_body(q_ref, v1_ref, ids1_ref, s2_ref, v2_ref, rid_ref, rs_ref):
    bq = q_ref.shape[0]
    d = q_ref.shape[1]
    q = q_ref[...]
    v1 = v1_ref[...].reshape(bq, TOPK, d)
    s2 = s2_ref[...]
    v2 = v2_ref[...].reshape(bq * TOPK, TOPK, d)

    top9 = jnp.max(v1[:, : TOPK - 1, :], axis=1)
    qtk = jnp.maximum(q, top9)
    qn = qtk * _rsqrt_nr(
        jnp.maximum(jnp.sum(qtk * qtk, axis=1, keepdims=True), 1e-24))

    w = s2 * BETA
    qrep = jnp.broadcast_to(q[:, None, :], (bq, TOPK, d)).reshape(bq * TOPK, d)
    ws = jnp.sum(v2 * w[:, :, None], axis=1) + BETA * qrep
    nf = 1.0 + BETA + jnp.sum(w, axis=1, keepdims=True)
    sgn = jnp.where(nf >= 0.0, 1.0, -1.0)
    inv = _rsqrt_nr(
        jnp.maximum(jnp.sum(ws * ws, axis=1, keepdims=True), 1e-24))
    refined = ws * (sgn * inv)

    def _bf(x):
        return x.astype(jnp.bfloat16).astype(jnp.float32)

    refb = _bf(refined)
    sc = 0.5 * (jnp.sum(refb * _bf(qrep), axis=1) +
                jnp.sum(refb.reshape(bq, TOPK, d) * _bf(qn)[:, None, :],
                        axis=2).reshape(bq * TOPK))
    sc = sc.reshape(bq, TOPK)

    ids1 = ids1_ref[...]
    mio = lax.broadcasted_iota(jnp.int32, (bq, TOPK), 1)
    vals, rids = [], []
    for _ in range(3):
        m = jnp.max(sc, axis=1, keepdims=True)
        pos = jnp.min(jnp.where(sc == m, mio, BIGI), axis=1, keepdims=True)
        vals.append(m)
        rids.append(jnp.sum(jnp.where(mio == pos, ids1, 0), axis=1,
                            keepdims=True))
        sc = jnp.where(mio == pos, NEG, sc)
    rs_ref[...] = jnp.concatenate(vals, axis=1)
    rid_ref[...] = jnp.concatenate(rids, axis=1)


def _epilogue(q, v1, ids1, s2, v2, bq=128, interpret=False):
    qn, d = q.shape
    grid = (qn // bq,)
    return pl.pallas_call(
        _epilogue_body,
        grid=grid,
        in_specs=[
            pl.BlockSpec((bq, d), lambda i: (i, 0)),
            pl.BlockSpec((bq * TOPK, d), lambda i: (i, 0)),
            pl.BlockSpec((bq, TOPK), lambda i: (i, 0)),
            pl.BlockSpec((bq * TOPK, TOPK), lambda i: (i, 0)),
            pl.BlockSpec((bq * TOPK * TOPK, d), lambda i: (i, 0)),
        ],
        out_specs=[
            pl.BlockSpec((bq, 3), lambda i: (i, 0)),
            pl.BlockSpec((bq, 3), lambda i: (i, 0)),
        ],
        out_shape=[
            jax.ShapeDtypeStruct((qn, 3), jnp.int32),
            jax.ShapeDtypeStruct((qn, 3), jnp.float32),
        ],
        interpret=interpret,
    )(q, v1, ids1, s2, v2)


def kernel(query_features, db):
    n, d = db.shape
    bn = 10240
    npad = ((n + bn - 1) // bn) * bn
    db_p = jnp.concatenate(
        [db, jnp.zeros((npad - n, d), db.dtype)], axis=0)

    _, ids1 = _topk(query_features, db_p, n)
    ids1_flat = ids1.reshape(-1)
    top_m = _sc_gather(db_p, ids1_flat)

    s2, ids2 = _topk(top_m, db_p, n)
    v2 = _sc_gather(db_p, ids2.reshape(-1))

    rid, rs = _epilogue(query_features, top_m, ids1, s2, v2)
    return (rid, rs)
